# Initial kernel scaffold; baseline (speedup 1.0000x reference)
import jax
import jax.numpy as jnp
from jax.experimental import pallas as pl


def kernel(x, edge_index, W, b):
    raise NotImplementedError("write your pallas kernel here")


# speedup vs baseline: 3.3412x; 3.3412x over previous
import functools

import jax
import jax.numpy as jnp
from jax import lax
from jax.experimental import pallas as pl
from jax.experimental.pallas import tpu as pltpu, tpu_sc as plsc

N_NODES = 10000
D = 128
NC = 2
NS = 16
NW = NC * NS
CH = 128
CPT = 80
E_PAD = NW * CPT * CH
SINK = N_NODES
ROWS_PER_TILE = 640
ROWS_SC = NS * ROWS_PER_TILE
BLK = 1000


def _fill1d_f32(ref, n, val):
    vec = jnp.full((16,), val, jnp.float32)

    def body(i, carry):
        ref[pl.ds(i * 16, 16)] = vec
        return carry

    lax.fori_loop(0, n // 16, body, 0)


def _fill2d_f32(ref, rows, cols, val):
    vec = jnp.full((16,), val, jnp.float32)
    ncol = cols // 16

    def body(i, carry):
        r = i // ncol
        c = (i % ncol) * 16
        ref[r, pl.ds(c, 16)] = vec
        return carry

    lax.fori_loop(0, rows * ncol, body, 0)


def _make_agg_kernel():
    mesh = plsc.VectorSubcoreMesh(core_axis_name="c", subcore_axis_name="s")

    @functools.partial(
        pl.kernel,
        mesh=mesh,
        out_type=[
            jax.ShapeDtypeStruct((NC, ROWS_SC, D), jnp.float32),
            jax.ShapeDtypeStruct((NC, ROWS_SC), jnp.float32),
        ],
        scratch_types=[
            pltpu.VMEM((CH,), jnp.int32),
            pltpu.VMEM((CH,), jnp.int32),
            pltpu.VMEM((CH, D), jnp.float32),
            pltpu.VMEM((CH,), jnp.float32),
            pltpu.VMEM((ROWS_PER_TILE,), jnp.float32),
            pltpu.VMEM_SHARED((ROWS_SC, D), jnp.float32),
            pltpu.VMEM_SHARED((ROWS_SC,), jnp.float32),
            pltpu.SemaphoreType.DMA,
        ],
    )
    def agg(x_hbm, src_hbm, dst_hbm, acc_out, deg_out,
            srcv, dstv, rb0, onesv, zdeg, acc_sh, deg_sh, sem0):
        c = lax.axis_index("c")
        s = lax.axis_index("s")
        w = c * NS + s

        _fill2d_f32(rb0, CH, D, 0.0)
        _fill1d_f32(onesv, CH, 1.0)
        _fill1d_f32(zdeg, ROWS_PER_TILE, 0.0)

        base = s * ROWS_PER_TILE
        for k in range(ROWS_PER_TILE // CH):
            pltpu.sync_copy(rb0, acc_sh.at[pl.ds(base + k * CH, CH)])
        pltpu.sync_copy(zdeg, deg_sh.at[pl.ds(base, ROWS_PER_TILE)])
        plsc.subcore_barrier()

        def body(j, carry):
            pltpu.sync_copy(src_hbm.at[w, j], srcv)
            pltpu.sync_copy(dst_hbm.at[w, j], dstv)
            pltpu.async_copy(x_hbm.at[srcv], rb0, sem0).wait()
            pltpu.sync_copy(rb0, acc_sh.at[dstv], add=True)
            pltpu.sync_copy(onesv, deg_sh.at[dstv], add=True)
            return carry

        lax.fori_loop(0, CPT, body, 0)
        plsc.subcore_barrier()

        pltpu.sync_copy(acc_sh.at[pl.ds(base, ROWS_PER_TILE)],
                        acc_out.at[c, pl.ds(base, ROWS_PER_TILE)])
        pltpu.sync_copy(deg_sh.at[pl.ds(base, ROWS_PER_TILE)],
                        deg_out.at[c, pl.ds(base, ROWS_PER_TILE)])

    return agg


_agg_kernel = _make_agg_kernel()


def _tc_body(acc_ref, deg_ref, w_ref, b_ref, o_ref):
    a = acc_ref[0] + acc_ref[1]
    dg = deg_ref[0] + deg_ref[1]
    m = a / jnp.maximum(dg, 1.0)
    h = lax.dot_general(m, w_ref[...], (((1,), (1,)), ((), ())),
                        preferred_element_type=jnp.float32)
    o_ref[...] = jnp.maximum(h + b_ref[...], 0.0)


def _tc_call(acc, deg, W, b2):
    grid = N_NODES // BLK
    return pl.pallas_call(
        _tc_body,
        grid=(grid,),
        in_specs=[
            pl.BlockSpec((NC, BLK, D), lambda i: (0, i, 0)),
            pl.BlockSpec((NC, BLK, 1), lambda i: (0, i, 0)),
            pl.BlockSpec((D, D), lambda i: (0, 0)),
            pl.BlockSpec((1, D), lambda i: (0, 0)),
        ],
        out_specs=pl.BlockSpec((BLK, D), lambda i: (i, 0)),
        out_shape=jax.ShapeDtypeStruct((N_NODES, D), jnp.float32),
    )(acc, deg, W, b2)


def kernel(x, edge_index, W, b):
    src = edge_index[0].astype(jnp.int32)
    dst = edge_index[1].astype(jnp.int32)
    e = src.shape[0]
    pad = E_PAD - e
    src_p = jnp.concatenate(
        [src, jnp.zeros((pad,), jnp.int32)]).reshape(NW, CPT, CH)
    dst_p = jnp.concatenate(
        [dst, jnp.full((pad,), SINK, jnp.int32)]).reshape(NW, CPT, CH)
    acc, deg = _agg_kernel(x, src_p, dst_p)
    return _tc_call(acc, deg.reshape(NC, ROWS_SC, 1), W, b.reshape(1, D))


# speedup vs baseline: 4.1223x; 1.2338x over previous
import functools

import jax
import jax.numpy as jnp
from jax import lax
from jax.experimental import pallas as pl
from jax.experimental.pallas import tpu as pltpu, tpu_sc as plsc

N_NODES = 10000
D = 128
NC = 2
NS = 16
NW = NC * NS
CH = 64
CPT = 160
NSLOT = 4
PF = 2
E_PAD = NW * CPT * CH
SINK = N_NODES
ROWS_PER_TILE = 640
ROWS_SC = NS * ROWS_PER_TILE
BLK = 1000


def _fill1d_f32(ref, n, val):
    vec = jnp.full((16,), val, jnp.float32)

    def body(i, carry):
        ref[pl.ds(i * 16, 16)] = vec
        return carry

    lax.fori_loop(0, n // 16, body, 0)


def _fill2d_f32(ref, rows, cols, val):
    vec = jnp.full((16,), val, jnp.float32)
    ncol = cols // 16

    def body(i, carry):
        r = i // ncol
        c = (i % ncol) * 16
        ref[r, pl.ds(c, 16)] = vec
        return carry

    lax.fori_loop(0, rows * ncol, body, 0)


def _make_agg_kernel():
    mesh = plsc.VectorSubcoreMesh(core_axis_name="c", subcore_axis_name="s")

    @functools.partial(
        pl.kernel,
        mesh=mesh,
        out_type=[
            jax.ShapeDtypeStruct((NC, ROWS_SC, D), jnp.float32),
            jax.ShapeDtypeStruct((NC, ROWS_SC), jnp.float32),
        ],
        scratch_types=(
            [pltpu.VMEM((CPT * CH,), jnp.int32)]
            + [pltpu.VMEM((CH,), jnp.int32) for _ in range(NSLOT)]
            + [pltpu.VMEM((CH, D), jnp.float32) for _ in range(NSLOT)]
            + [pltpu.VMEM((CH,), jnp.float32)]
            + [pltpu.VMEM((ROWS_PER_TILE,), jnp.float32)]
            + [pltpu.VMEM_SHARED((ROWS_SC, D), jnp.float32)]
            + [pltpu.VMEM_SHARED((ROWS_SC,), jnp.float32)]
            + [pltpu.SemaphoreType.DMA for _ in range(3 * NSLOT)]
        ),
    )
    def agg(x_hbm, src_hbm, dst_hbm, acc_out, deg_out, src_all,
            d0, d1, d2, d3, r0, r1, r2, r3, onesv, zdeg,
            acc_sh, deg_sh,
            i0, i1, i2, i3, g0, g1, g2, g3, s0, s1, s2, s3):
        dstv = [d0, d1, d2, d3]
        rows = [r0, r1, r2, r3]
        isem = [i0, i1, i2, i3]
        gsem = [g0, g1, g2, g3]
        ssem = [s0, s1, s2, s3]
        c = lax.axis_index("c")
        s = lax.axis_index("s")
        w = c * NS + s

        pltpu.sync_copy(src_hbm.at[w], src_all)

        _fill2d_f32(r0, CH, D, 0.0)
        _fill1d_f32(onesv, CH, 1.0)
        _fill1d_f32(zdeg, ROWS_PER_TILE, 0.0)
        base = s * ROWS_PER_TILE
        for k in range(ROWS_PER_TILE // CH):
            pltpu.sync_copy(r0, acc_sh.at[pl.ds(base + k * CH, CH)])
        pltpu.sync_copy(zdeg, deg_sh.at[pl.ds(base, ROWS_PER_TILE)])
        plsc.subcore_barrier()

        def issue_fetch(j, b):
            pltpu.async_copy(dst_hbm.at[w, j], dstv[b], isem[b])
            pltpu.async_copy(x_hbm.at[src_all.at[pl.ds(j * CH, CH)]],
                             rows[b], gsem[b])

        def wait_scatter(b):
            pltpu.make_async_copy(rows[b], acc_sh.at[dstv[b]], ssem[b]).wait()
            pltpu.make_async_copy(onesv, deg_sh.at[dstv[b]], ssem[b]).wait()

        for b in range(PF):
            issue_fetch(b, b)

        def body(g, carry):
            for b in range(NSLOT):
                j = g * NSLOT + b
                pltpu.make_async_copy(dst_hbm.at[w, j], dstv[b],
                                      isem[b]).wait()
                pltpu.make_async_copy(
                    x_hbm.at[src_all.at[pl.ds(j * CH, CH)]],
                    rows[b], gsem[b]).wait()
                pltpu.async_copy(rows[b], acc_sh.at[dstv[b]], ssem[b],
                                 add=True)
                pltpu.async_copy(onesv, deg_sh.at[dstv[b]], ssem[b],
                                 add=True)

                bn = (b + PF) % NSLOT

                @pl.when(j < CPT - PF)
                def _():
                    if b + PF >= NSLOT:
                        wait_scatter(bn)
                        issue_fetch(j + PF, bn)
                    else:
                        @pl.when(g >= 1)
                        def _():
                            wait_scatter(bn)
                        issue_fetch(j + PF, bn)
            return carry

        lax.fori_loop(0, CPT // NSLOT, body, 0)

        for b in range(NSLOT):
            wait_scatter(b)
        plsc.subcore_barrier()

        pltpu.sync_copy(acc_sh.at[pl.ds(base, ROWS_PER_TILE)],
                        acc_out.at[c, pl.ds(base, ROWS_PER_TILE)])
        pltpu.sync_copy(deg_sh.at[pl.ds(base, ROWS_PER_TILE)],
                        deg_out.at[c, pl.ds(base, ROWS_PER_TILE)])

    return agg


_agg_kernel = _make_agg_kernel()


def _tc_body(acc_ref, deg_ref, w_ref, b_ref, o_ref):
    a = acc_ref[0] + acc_ref[1]
    dg = deg_ref[0] + deg_ref[1]
    m = a / jnp.maximum(dg, 1.0)
    h = lax.dot_general(m, w_ref[...], (((1,), (1,)), ((), ())),
                        preferred_element_type=jnp.float32)
    o_ref[...] = jnp.maximum(h + b_ref[...], 0.0)


def _tc_call(acc, deg, W, b2):
    grid = N_NODES // BLK
    return pl.pallas_call(
        _tc_body,
        grid=(grid,),
        in_specs=[
            pl.BlockSpec((NC, BLK, D), lambda i: (0, i, 0)),
            pl.BlockSpec((NC, BLK, 1), lambda i: (0, i, 0)),
            pl.BlockSpec((D, D), lambda i: (0, 0)),
            pl.BlockSpec((1, D), lambda i: (0, 0)),
        ],
        out_specs=pl.BlockSpec((BLK, D), lambda i: (i, 0)),
        out_shape=jax.ShapeDtypeStruct((N_NODES, D), jnp.float32),
    )(acc, deg, W, b2)


def kernel(x, edge_index, W, b):
    src = edge_index[0].astype(jnp.int32)
    dst = edge_index[1].astype(jnp.int32)
    e = src.shape[0]
    pad = E_PAD - e
    src_p = jnp.concatenate(
        [src, jnp.zeros((pad,), jnp.int32)]).reshape(NW, CPT * CH)
    dst_p = jnp.concatenate(
        [dst, jnp.full((pad,), SINK, jnp.int32)]).reshape(NW, CPT, CH)
    acc, deg = _agg_kernel(x, src_p, dst_p)
    return _tc_call(acc, deg.reshape(NC, ROWS_SC, 1), W, b.reshape(1, D))


# speedup vs baseline: 14.2501x; 3.4569x over previous
import functools

import jax
import jax.numpy as jnp
from jax import lax
from jax.experimental import pallas as pl
from jax.experimental.pallas import tpu as pltpu, tpu_sc as plsc

N_NODES = 10000
D = 128
NC = 2
NS = 16
NW = NC * NS
N_EDGES = 320000
EPT = N_EDGES // NW
CH = 64
CPT = EPT // CH
TAIL = EPT - CPT * CH
NSLOT = 4
PF = 2
ROWS_PER_TILE = 640
ROWS_SC = NS * ROWS_PER_TILE
BLK = 1000


def _fill1d_f32(ref, n, val):
    vec = jnp.full((16,), val, jnp.float32)

    def body(i, carry):
        ref[pl.ds(i * 16, 16)] = vec
        return carry

    lax.fori_loop(0, n // 16, body, 0)


def _fill2d_f32(ref, rows, cols, val):
    vec = jnp.full((16,), val, jnp.float32)
    ncol = cols // 16

    def body(i, carry):
        r = i // ncol
        c = (i % ncol) * 16
        ref[r, pl.ds(c, 16)] = vec
        return carry

    lax.fori_loop(0, rows * ncol, body, 0)


def _make_agg_kernel():
    mesh = plsc.VectorSubcoreMesh(core_axis_name="c", subcore_axis_name="s")

    @functools.partial(
        pl.kernel,
        mesh=mesh,
        out_type=[
            jax.ShapeDtypeStruct((NC, ROWS_SC, D), jnp.float32),
            jax.ShapeDtypeStruct((NC, ROWS_SC), jnp.float32),
        ],
        scratch_types=(
            [pltpu.VMEM((EPT,), jnp.int32)]
            + [pltpu.VMEM((CH,), jnp.int32) for _ in range(NSLOT)]
            + [pltpu.VMEM((CH, D), jnp.float32) for _ in range(NSLOT)]
            + [pltpu.VMEM((CH,), jnp.float32)]
            + [pltpu.VMEM((TAIL,), jnp.int32)]
            + [pltpu.VMEM((TAIL, D), jnp.float32)]
            + [pltpu.VMEM((TAIL,), jnp.float32)]
            + [pltpu.VMEM((ROWS_PER_TILE,), jnp.float32)]
            + [pltpu.VMEM_SHARED((ROWS_SC, D), jnp.float32)]
            + [pltpu.VMEM_SHARED((ROWS_SC,), jnp.float32)]
            + [pltpu.SemaphoreType.DMA for _ in range(3 * NSLOT)]
        ),
    )
    def agg(x_hbm, edge_hbm, acc_out, deg_out, src_all,
            d0, d1, d2, d3, r0, r1, r2, r3, onesv, tdst, trows, tones, zdeg,
            acc_sh, deg_sh,
            i0, i1, i2, i3, g0, g1, g2, g3, s0, s1, s2, s3):
        dstv = [d0, d1, d2, d3]
        rows = [r0, r1, r2, r3]
        isem = [i0, i1, i2, i3]
        gsem = [g0, g1, g2, g3]
        ssem = [s0, s1, s2, s3]
        c = lax.axis_index("c")
        s = lax.axis_index("s")
        w = c * NS + s
        e0 = w * EPT

        pltpu.sync_copy(edge_hbm.at[pl.ds(e0, EPT)], src_all)

        _fill2d_f32(r0, CH, D, 0.0)
        _fill1d_f32(onesv, CH, 1.0)
        _fill1d_f32(tones, TAIL, 1.0)
        _fill1d_f32(zdeg, ROWS_PER_TILE, 0.0)
        base = s * ROWS_PER_TILE
        for k in range(ROWS_PER_TILE // CH):
            pltpu.sync_copy(r0, acc_sh.at[pl.ds(base + k * CH, CH)])
        pltpu.sync_copy(zdeg, deg_sh.at[pl.ds(base, ROWS_PER_TILE)])
        plsc.subcore_barrier()

        def issue_fetch(j, b):
            pltpu.async_copy(edge_hbm.at[pl.ds(N_EDGES + e0 + j * CH, CH)],
                             dstv[b], isem[b])
            pltpu.async_copy(x_hbm.at[src_all.at[pl.ds(j * CH, CH)]],
                             rows[b], gsem[b])

        def wait_scatter(b):
            pltpu.make_async_copy(rows[b], acc_sh.at[dstv[b]], ssem[b]).wait()
            pltpu.make_async_copy(onesv, deg_sh.at[dstv[b]], ssem[b]).wait()

        for b in range(PF):
            issue_fetch(b, b)

        def body(g, carry):
            for b in range(NSLOT):
                j = g * NSLOT + b
                pltpu.make_async_copy(
                    edge_hbm.at[pl.ds(N_EDGES + e0 + j * CH, CH)],
                    dstv[b], isem[b]).wait()
                pltpu.make_async_copy(
                    x_hbm.at[src_all.at[pl.ds(j * CH, CH)]],
                    rows[b], gsem[b]).wait()
                pltpu.async_copy(rows[b], acc_sh.at[dstv[b]], ssem[b],
                                 add=True)
                pltpu.async_copy(onesv, deg_sh.at[dstv[b]], ssem[b],
                                 add=True)

                bn = (b + PF) % NSLOT

                @pl.when(j < CPT - PF)
                def _():
                    if b + PF >= NSLOT:
                        wait_scatter(bn)
                        issue_fetch(j + PF, bn)
                    else:
                        @pl.when(g >= 1)
                        def _():
                            wait_scatter(bn)
                        issue_fetch(j + PF, bn)
            return carry

        lax.fori_loop(0, CPT // NSLOT, body, 0)

        for b in range(NSLOT):
            wait_scatter(b)

        t0 = N_EDGES + e0 + CPT * CH
        pltpu.sync_copy(edge_hbm.at[pl.ds(t0, TAIL)], tdst)
        pltpu.sync_copy(x_hbm.at[src_all.at[pl.ds(CPT * CH, TAIL)]], trows)
        pltpu.sync_copy(trows, acc_sh.at[tdst], add=True)
        pltpu.sync_copy(tones, deg_sh.at[tdst], add=True)
        plsc.subcore_barrier()

        pltpu.sync_copy(acc_sh.at[pl.ds(base, ROWS_PER_TILE)],
                        acc_out.at[c, pl.ds(base, ROWS_PER_TILE)])
        pltpu.sync_copy(deg_sh.at[pl.ds(base, ROWS_PER_TILE)],
                        deg_out.at[c, pl.ds(base, ROWS_PER_TILE)])

    return agg


_agg_kernel = _make_agg_kernel()


def _tc_body(acc_ref, deg_ref, w_ref, b_ref, o_ref):
    a = acc_ref[0] + acc_ref[1]
    dg = deg_ref[0] + deg_ref[1]
    m = a / jnp.maximum(dg, 1.0)
    h = lax.dot_general(m, w_ref[...], (((1,), (1,)), ((), ())),
                        preferred_element_type=jnp.float32)
    o_ref[...] = jnp.maximum(h + b_ref[...], 0.0)


def _tc_call(acc, deg, W, b2):
    grid = N_NODES // BLK
    return pl.pallas_call(
        _tc_body,
        grid=(grid,),
        in_specs=[
            pl.BlockSpec((NC, BLK, D), lambda i: (0, i, 0)),
            pl.BlockSpec((NC, BLK, 1), lambda i: (0, i, 0)),
            pl.BlockSpec((D, D), lambda i: (0, 0)),
            pl.BlockSpec((1, D), lambda i: (0, 0)),
        ],
        out_specs=pl.BlockSpec((BLK, D), lambda i: (i, 0)),
        out_shape=jax.ShapeDtypeStruct((N_NODES, D), jnp.float32),
    )(acc, deg, W, b2)


def kernel(x, edge_index, W, b):
    acc, deg = _agg_kernel(x, edge_index.astype(jnp.int32).reshape(-1))
    return _tc_call(acc, deg.reshape(NC, ROWS_SC, 1), W, b.reshape(1, D))
